# Initial kernel scaffold; baseline (speedup 1.0000x reference)
import functools

import numpy as np
import jax
import jax.numpy as jnp
from jax import lax
from jax.experimental import pallas as pl
from jax.experimental.pallas import tpu as pltpu
from jax.experimental.pallas import tpu_sc as plsc

N = 1024
E = 16384
D = 128
M = 32
MAXN = 30
KNEI = 5
BIG = 1e9

NC = 2
NS = 16
NW = NC * NS
RW = N // NW

_ORDER_NP = None


def _get_order():
    global _ORDER_NP
    if _ORDER_NP is None:
        cpu = jax.devices("cpu")[0]
        with jax.default_device(cpu):
            key = jax.random.key(42)
            scores = jax.random.uniform(key, (N, N))
            diag = jnp.arange(N)
            scores = scores.at[diag, diag].set(jnp.inf)
            order = jnp.argsort(-scores, axis=1, stable=True)
        _ORDER_NP = np.asarray(order).astype(np.int32)
    return _ORDER_NP


def _adj_body(edges_hbm, zero_hbm, a_hbm, ebuf, arow):
    c = lax.axis_index("c")
    s = lax.axis_index("s")
    wid = s * NC + c
    base = wid * RW

    pltpu.sync_copy(zero_hbm, arow)
    pltpu.sync_copy(edges_hbm, ebuf)

    onesf = jnp.full((16,), 1.0, jnp.float32)

    def ebody(i, carry):
        aa = ebuf[0, pl.ds(i * 16, 16)]
        bb = ebuf[1, pl.ds(i * 16, 16)]
        ra = aa - base
        ma = (ra >= 0) & (ra < RW)
        ra_c = jnp.where(ma, ra, 0)
        plsc.store_scatter(arow, [ra_c, bb], onesf, mask=ma)
        rb = bb - base
        mb = (rb >= 0) & (rb < RW)
        rb_c = jnp.where(mb, rb, 0)
        plsc.store_scatter(arow, [rb_c, aa], onesf, mask=mb)
        return carry

    lax.fori_loop(0, E // 16, ebody, 0)

    iota16 = lax.iota(jnp.int32, 16)
    for ch in range(RW // 16):
        rv = iota16 + ch * 16
        plsc.store_scatter(arow, [rv, base + rv], onesf)

    pltpu.sync_copy(arow, a_hbm.at[pl.ds(base, RW)])


def _build_adj(edge_index):
    mesh = plsc.VectorSubcoreMesh(core_axis_name="c", subcore_axis_name="s")
    zero = jnp.zeros((RW, N), jnp.float32)
    kfn = functools.partial(
        pl.kernel,
        mesh=mesh,
        out_type=jax.ShapeDtypeStruct((N, N), jnp.float32),
        scratch_types=[
            pltpu.VMEM((2, E), jnp.int32),
            pltpu.VMEM((RW, N), jnp.float32),
        ],
    )(_adj_body)
    return kfn(edge_index, zero)


def _mask_body(a_ref, m_ref):
    ab = a_ref[...].astype(jnp.bfloat16)
    cmat = lax.dot_general(ab, ab, (((1,), (0,)), ((), ())),
                           preferred_element_type=jnp.float32)
    m_ref[...] = (cmat > 0.0).astype(jnp.float32)


def _mask_tc(a):
    return pl.pallas_call(
        _mask_body,
        out_shape=jax.ShapeDtypeStruct((N, N), jnp.float32),
    )(a)


def _select_body(order_hbm, mask_hbm, h_hbm, hl_hbm, scnt_hbm,
                 morow, orow, idxb, scntv, grow, sem):
    c = lax.axis_index("c")
    s = lax.axis_index("s")
    wid = s * NC + c
    base = wid * RW

    pltpu.sync_copy(mask_hbm.at[pl.ds(base, RW)], morow)
    pltpu.sync_copy(order_hbm.at[pl.ds(base, RW)], orow)

    iota16 = lax.iota(jnp.int32, 16)
    lane0 = iota16 == 0

    def row_body(r, carry):
        selfv = jnp.broadcast_to(base + r, (16,)).astype(jnp.int32)
        idxb[pl.ds(r * M, 16)] = selfv
        idxb[pl.ds(r * M + 16, 16)] = selfv

        def chunk_body(k, found):
            ordv = orow[r, pl.ds(k * 16, 16)]
            rsp = jnp.broadcast_to(r, (16,)).astype(jnp.int32)
            mp = plsc.load_gather(morow, [rsp, ordv])
            mb = mp > 0.0
            cs = plsc.cumsum(mb.astype(jnp.int32))
            sel = mb & ((found + cs) <= MAXN)
            plsc.store_compressed(idxb.at[pl.ds(r * M + found, 16)], ordv, sel)
            return found + jnp.sum(sel.astype(jnp.int32))

        found = lax.fori_loop(0, N // 16, chunk_body, jnp.int32(0))
        rsp = jnp.broadcast_to(r, (16,)).astype(jnp.int32)
        fsp = jnp.broadcast_to(found, (16,)).astype(jnp.int32)
        plsc.store_scatter(scntv, [rsp], fsp, mask=lane0)
        return carry

    lax.fori_loop(0, RW, row_body, 0)
    pltpu.sync_copy(scntv, scnt_hbm.at[pl.ds(base, RW)])

    def gath_body(r, carry):
        pltpu.async_copy(h_hbm.at[idxb.at[pl.ds(r * M, M)]], grow, sem).wait()
        pltpu.sync_copy(grow, hl_hbm.at[base + r])
        return carry

    lax.fori_loop(0, RW, gath_body, 0)


def _select_gather(order, maskf, H):
    mesh = plsc.VectorSubcoreMesh(core_axis_name="c", subcore_axis_name="s")
    kfn = functools.partial(
        pl.kernel,
        mesh=mesh,
        out_type=(
            jax.ShapeDtypeStruct((N, M, D), jnp.float32),
            jax.ShapeDtypeStruct((N,), jnp.int32),
        ),
        scratch_types=[
            pltpu.VMEM((RW, N), jnp.float32),
            pltpu.VMEM((RW, N), jnp.int32),
            pltpu.VMEM((RW * M + 16,), jnp.int32),
            pltpu.VMEM((RW,), jnp.int32),
            pltpu.VMEM((M, D), jnp.float32),
            pltpu.SemaphoreType.DMA,
        ],
    )(_select_body)
    return kfn(order, maskf, H)


_BB = 8


def _stats_body(hl_ref, scnt_ref, out_ref):
    G = hl_ref[...]
    scnt = scnt_ref[0, 0, :]
    sq = jnp.sum(G * G, axis=-1)
    inner = lax.dot_general(G, G, (((2,), (2,)), ((0,), (0,))),
                            preferred_element_type=jnp.float32)
    d2 = sq[:, :, None] + sq[:, None, :] - 2.0 * inner
    d2 = jnp.maximum(d2, 0.0)
    dist = jnp.sqrt(d2 + 1e-12)

    mi = lax.broadcasted_iota(jnp.int32, (_BB, M, M), 1)
    ki = lax.broadcasted_iota(jnp.int32, (_BB, M, M), 2)
    sc3 = scnt[:, None, None]
    off = (mi < sc3) & (ki < sc3) & (mi != ki)
    dist_m = jnp.where(off, dist, BIG)

    knn = []
    cur = dist_m
    for _ in range(KNEI):
        mn = jnp.min(cur, axis=-1)
        eq = cur == mn[:, :, None]
        li = jnp.min(jnp.where(eq, ki, N * M), axis=-1)
        cur = jnp.where(ki == li[:, :, None], BIG, cur)
        knn.append(mn)
    knn_d = jnp.stack(knn, axis=-1)

    valid2 = lax.broadcasted_iota(jnp.int32, (_BB, M), 1) < scnt[:, None]
    kslot = lax.broadcasted_iota(jnp.int32, (_BB, M, KNEI), 2)
    kw = (kslot < (scnt[:, None, None] - 1)) & valid2[:, :, None]
    kwf = kw.astype(jnp.float32)
    knn_d = jnp.where(kw, knn_d, 0.0)

    n_k = jnp.sum(kwf, axis=(1, 2)) + 1e-12
    sum_knn = jnp.sum(knn_d, axis=(1, 2))
    mean_knn = sum_knn / n_k
    max_knn = jnp.max(knn_d, axis=(1, 2))
    dev = (knn_d - mean_knn[:, None, None]) * kwf
    std_knn = jnp.sqrt(jnp.sum(dev * dev, axis=(1, 2)) / n_k + 1e-12)

    pd = jnp.where(off, dist, 0.0)
    scf = scnt.astype(jnp.float32)
    n_pd = scf * scf - scf + 1e-12
    mean_pd = jnp.sum(pd, axis=(1, 2)) / n_pd

    p = knn_d / (sum_knn[:, None, None] + 1e-12)
    ent = -jnp.sum(p * jnp.log(p + 1e-12) * kwf, axis=(1, 2))

    sigs = jnp.stack([mean_knn, max_knn, sum_knn, std_knn, mean_pd, ent],
                     axis=-1)
    sigs = jnp.where((scnt >= 3)[:, None], sigs, 0.0)
    out_ref[...] = jnp.concatenate(
        [sigs, jnp.zeros((_BB, 2), jnp.float32)], axis=-1)


def _stats_tc(hl, scnt3):
    return pl.pallas_call(
        _stats_body,
        grid=(N // _BB,),
        in_specs=[
            pl.BlockSpec((_BB, M, D), lambda i: (i, 0, 0)),
            pl.BlockSpec((1, 1, _BB), lambda i: (i, 0, 0)),
        ],
        out_specs=pl.BlockSpec((_BB, 8), lambda i: (i, 0)),
        out_shape=jax.ShapeDtypeStruct((N, 8), jnp.float32),
    )(hl, scnt3)


def kernel(H, edge_index):
    order = _get_order()
    a = _build_adj(edge_index)
    maskf = _mask_tc(a)
    hl, scnt = _select_gather(jnp.asarray(order), maskf, H)
    sigs8 = _stats_tc(hl, scnt.reshape(N // _BB, 1, _BB))
    return sigs8[:, :6]


# speedup vs baseline: 3.2777x; 3.2777x over previous
import functools

import numpy as np
import jax
import jax.numpy as jnp
from jax import lax
from jax.experimental import pallas as pl
from jax.experimental.pallas import tpu as pltpu
from jax.experimental.pallas import tpu_sc as plsc

N = 1024
E = 16384
D = 128
M = 32
MAXN = 30
KNEI = 5
BIG = 1e9

NC = 2
NS = 16
NW = NC * NS
RW = N // NW

_ORDER_NP = None


def _threefry2x32(k0, k1, x0, x1):
    def rotl(x, r):
        return ((x << np.uint32(r)) | (x >> np.uint32(32 - r))).astype(np.uint32)

    ks = [np.uint32(k0), np.uint32(k1),
          np.uint32(k0 ^ k1 ^ np.uint32(0x1BD11BDA))]
    rotations = [[13, 15, 26, 6], [17, 29, 16, 24]]
    x0 = (x0 + ks[0]).astype(np.uint32)
    x1 = (x1 + ks[1]).astype(np.uint32)
    for i in range(5):
        for r in rotations[i % 2]:
            x0 = (x0 + x1).astype(np.uint32)
            x1 = rotl(x1, r)
            x1 = x1 ^ x0
        x0 = (x0 + ks[(i + 1) % 3]).astype(np.uint32)
        x1 = (x1 + ks[(i + 2) % 3] + np.uint32(i + 1)).astype(np.uint32)
    return x0, x1


def _uniform_scores():
    size = N * N
    with np.errstate(over="ignore"):
        cnt = np.arange(size, dtype=np.uint32)
        h0, h1 = _threefry2x32(0, 42, np.zeros(size, np.uint32), cnt)
    bits = h0 ^ h1
    fl = ((bits >> np.uint32(9)) | np.uint32(0x3F800000)).view(np.float32)
    u = np.maximum(np.float32(0.0), fl - np.float32(1.0))
    return u.reshape(N, N)


def _get_order():
    global _ORDER_NP
    if _ORDER_NP is None:
        scores = _uniform_scores()
        np.fill_diagonal(scores, np.inf)
        _ORDER_NP = np.argsort(-scores, axis=1, kind="stable").astype(np.int32)
    return _ORDER_NP


def _adj_body(edges_hbm, zero_hbm, a_hbm, ebuf, arow):
    c = lax.axis_index("c")
    s = lax.axis_index("s")
    wid = s * NC + c
    base = wid * RW

    pltpu.sync_copy(zero_hbm, arow)
    pltpu.sync_copy(edges_hbm, ebuf)

    onesf = jnp.full((16,), 1.0, jnp.float32)

    def ebody(i, carry):
        aa = ebuf[pl.ds(i * 16, 16)]
        bb = ebuf[pl.ds(E + i * 16, 16)]
        ra = aa - base
        ma = (ra >= 0) & (ra < RW)
        ra_c = jnp.where(ma, ra, 0)
        plsc.store_scatter(arow, [ra_c * N + bb], onesf, mask=ma)
        rb = bb - base
        mb = (rb >= 0) & (rb < RW)
        rb_c = jnp.where(mb, rb, 0)
        plsc.store_scatter(arow, [rb_c * N + aa], onesf, mask=mb)
        return carry

    lax.fori_loop(0, E // 16, ebody, 0)

    iota16 = lax.iota(jnp.int32, 16)
    for ch in range(RW // 16):
        rv = iota16 + ch * 16
        plsc.store_scatter(arow, [rv * N + base + rv], onesf)

    pltpu.sync_copy(arow, a_hbm.at[pl.ds(base * N, RW * N)])


def _build_adj(edge_index):
    mesh = plsc.VectorSubcoreMesh(core_axis_name="c", subcore_axis_name="s")
    zero = jnp.zeros((RW * N,), jnp.float32)
    kfn = functools.partial(
        pl.kernel,
        mesh=mesh,
        compiler_params=pltpu.CompilerParams(needs_layout_passes=False),
        out_type=jax.ShapeDtypeStruct((N * N,), jnp.float32),
        scratch_types=[
            pltpu.VMEM((2 * E,), jnp.int32),
            pltpu.VMEM((RW * N,), jnp.float32),
        ],
    )(_adj_body)
    return kfn(edge_index.reshape(2 * E), zero).reshape(N, N)


def _mask_body(a_ref, m_ref):
    ab = a_ref[...].astype(jnp.bfloat16)
    cmat = lax.dot_general(ab, ab, (((1,), (0,)), ((), ())),
                           preferred_element_type=jnp.float32)
    m_ref[...] = (cmat > 0.0).astype(jnp.float32)


def _mask_tc(a):
    return pl.pallas_call(
        _mask_body,
        out_shape=jax.ShapeDtypeStruct((N, N), jnp.float32),
    )(a)


def _select_body(order_hbm, mask_hbm, h_hbm, hl_hbm, scnt_hbm,
                 morow, orow, idxb, scntv, grow, sem):
    c = lax.axis_index("c")
    s = lax.axis_index("s")
    wid = s * NC + c
    base = wid * RW

    pltpu.sync_copy(mask_hbm.at[pl.ds(base * N, RW * N)], morow)
    pltpu.sync_copy(order_hbm.at[pl.ds(base * N, RW * N)], orow)

    iota16 = lax.iota(jnp.int32, 16)
    lane0 = iota16 == 0

    def row_body(r, carry):
        selfv = jnp.broadcast_to(base + r, (16,)).astype(jnp.int32)
        idxb[pl.ds(r * M, 16)] = selfv
        idxb[pl.ds(r * M + 16, 16)] = selfv

        def chunk_body(k, found):
            ordv = orow[pl.ds(r * N + k * 16, 16)]
            mp = plsc.load_gather(morow, [r * N + ordv])
            mb = mp > 0.0
            cs = plsc.cumsum(mb.astype(jnp.int32))
            sel = mb & ((found + cs) <= MAXN)
            plsc.store_compressed(idxb.at[pl.ds(r * M + found, 16)], ordv,
                                  mask=sel)
            return found + jnp.sum(sel.astype(jnp.int32))

        found = lax.fori_loop(0, N // 16, chunk_body, jnp.int32(0))
        rsp = jnp.broadcast_to(r, (16,)).astype(jnp.int32)
        fsp = jnp.broadcast_to(found, (16,)).astype(jnp.int32)
        plsc.store_scatter(scntv, [rsp], fsp, mask=lane0)
        return carry

    lax.fori_loop(0, RW, row_body, 0)
    pltpu.sync_copy(scntv, scnt_hbm.at[pl.ds(base, RW)])

    def gath_body(r, carry):
        pltpu.async_copy(h_hbm.at[idxb.at[pl.ds(r * M, M)]], grow, sem).wait()
        pltpu.sync_copy(grow, hl_hbm.at[base + r])
        return carry

    lax.fori_loop(0, RW, gath_body, 0)


def _select_gather(order, maskf, H):
    mesh = plsc.VectorSubcoreMesh(core_axis_name="c", subcore_axis_name="s")
    kfn = functools.partial(
        pl.kernel,
        mesh=mesh,
        compiler_params=pltpu.CompilerParams(needs_layout_passes=False),
        out_type=(
            jax.ShapeDtypeStruct((N, M, D), jnp.float32),
            jax.ShapeDtypeStruct((N,), jnp.int32),
        ),
        scratch_types=[
            pltpu.VMEM((RW * N,), jnp.float32),
            pltpu.VMEM((RW * N,), jnp.int32),
            pltpu.VMEM((RW * M + 16,), jnp.int32),
            pltpu.VMEM((RW,), jnp.int32),
            pltpu.VMEM((M, D), jnp.float32),
            pltpu.SemaphoreType.DMA,
        ],
    )(_select_body)
    return kfn(order.reshape(N * N), maskf.reshape(N * N), H)


_BB = 8


def _bcast(x, shape, dims):
    return lax.broadcast_in_dim(x, shape, dims)


def _stats_body(hl_ref, scnt_ref, out_ref):
    G = hl_ref[...]
    scnt = scnt_ref[0, 0, :]
    sq = jnp.sum(G * G, axis=-1)
    inner = lax.dot_general(G, G, (((2,), (2,)), ((0,), (0,))),
                            preferred_element_type=jnp.float32)
    d2 = (_bcast(sq, (_BB, M, M), (0, 1)) + _bcast(sq, (_BB, M, M), (0, 2))
          - 2.0 * inner)
    d2 = jnp.maximum(d2, 0.0)
    dist = jnp.sqrt(d2 + 1e-12)

    mi = lax.broadcasted_iota(jnp.int32, (_BB, M, M), 1)
    ki = lax.broadcasted_iota(jnp.int32, (_BB, M, M), 2)
    sc3 = _bcast(scnt, (_BB, M, M), (0,))
    off = (mi < sc3) & (ki < sc3) & (mi != ki)
    dist_m = jnp.where(off, dist, BIG)

    knn = []
    cur = dist_m
    for _ in range(KNEI):
        mn = jnp.min(cur, axis=-1)
        eq = cur == _bcast(mn, (_BB, M, M), (0, 1))
        li = jnp.min(jnp.where(eq, ki, N * M), axis=-1)
        cur = jnp.where(ki == _bcast(li, (_BB, M, M), (0, 1)), BIG, cur)
        knn.append(mn)
    knn_d = jnp.stack(knn, axis=-1)

    sck = _bcast(scnt, (_BB, M, KNEI), (0,))
    mik = lax.broadcasted_iota(jnp.int32, (_BB, M, KNEI), 1)
    kslot = lax.broadcasted_iota(jnp.int32, (_BB, M, KNEI), 2)
    kw = (kslot < (sck - 1)) & (mik < sck)
    kwf = kw.astype(jnp.float32)
    knn_d = jnp.where(kw, knn_d, 0.0)

    n_k = jnp.sum(kwf, axis=(1, 2)) + 1e-12
    sum_knn = jnp.sum(knn_d, axis=(1, 2))
    mean_knn = sum_knn / n_k
    max_knn = jnp.max(knn_d, axis=(1, 2))
    dev = (knn_d - _bcast(mean_knn, (_BB, M, KNEI), (0,))) * kwf
    std_knn = jnp.sqrt(jnp.sum(dev * dev, axis=(1, 2)) / n_k + 1e-12)

    pd = jnp.where(off, dist, 0.0)
    scf = scnt.astype(jnp.float32)
    n_pd = scf * scf - scf + 1e-12
    mean_pd = jnp.sum(pd, axis=(1, 2)) / n_pd

    p = knn_d / (_bcast(sum_knn, (_BB, M, KNEI), (0,)) + 1e-12)
    ent = -jnp.sum(p * jnp.log(p + 1e-12) * kwf, axis=(1, 2))

    sigs = jnp.stack([mean_knn, max_knn, sum_knn, std_knn, mean_pd, ent],
                     axis=-1)
    gf = (scnt >= 3).astype(jnp.float32)
    sigs = sigs * _bcast(gf, (_BB, 6), (0,))
    out_ref[...] = jnp.concatenate(
        [sigs, jnp.zeros((_BB, 2), jnp.float32)], axis=-1)


def _stats_tc(hl, scnt3):
    return pl.pallas_call(
        _stats_body,
        grid=(N // _BB,),
        in_specs=[
            pl.BlockSpec((_BB, M, D), lambda i: (i, 0, 0)),
            pl.BlockSpec((1, 1, _BB), lambda i: (i, 0, 0)),
        ],
        out_specs=pl.BlockSpec((_BB, 8), lambda i: (i, 0)),
        out_shape=jax.ShapeDtypeStruct((N, 8), jnp.float32),
    )(hl, scnt3)


def kernel(H, edge_index):
    order = _get_order()
    a = _build_adj(edge_index)
    maskf = _mask_tc(a)
    hl, scnt = _select_gather(jnp.asarray(order), maskf, H)
    sigs8 = _stats_tc(hl, scnt.reshape(N // _BB, 1, _BB))
    return sigs8[:, :6]


# speedup vs baseline: 5.6245x; 1.7160x over previous
import functools

import numpy as np
import jax
import jax.numpy as jnp
from jax import lax
from jax.experimental import pallas as pl
from jax.experimental.pallas import tpu as pltpu
from jax.experimental.pallas import tpu_sc as plsc

N = 1024
E = 16384
D = 128
M = 32
MAXN = 30
KNEI = 5
BIG = 1e9

NC = 2
NS = 16
NW = NC * NS
RW = N // NW

_ORDER_NP = None


def _threefry2x32(k0, k1, x0, x1):
    def rotl(x, r):
        return ((x << np.uint32(r)) | (x >> np.uint32(32 - r))).astype(np.uint32)

    ks = [np.uint32(k0), np.uint32(k1),
          np.uint32(k0 ^ k1 ^ np.uint32(0x1BD11BDA))]
    rotations = [[13, 15, 26, 6], [17, 29, 16, 24]]
    x0 = (x0 + ks[0]).astype(np.uint32)
    x1 = (x1 + ks[1]).astype(np.uint32)
    for i in range(5):
        for r in rotations[i % 2]:
            x0 = (x0 + x1).astype(np.uint32)
            x1 = rotl(x1, r)
            x1 = x1 ^ x0
        x0 = (x0 + ks[(i + 1) % 3]).astype(np.uint32)
        x1 = (x1 + ks[(i + 2) % 3] + np.uint32(i + 1)).astype(np.uint32)
    return x0, x1


def _uniform_scores():
    size = N * N
    with np.errstate(over="ignore"):
        cnt = np.arange(size, dtype=np.uint32)
        h0, h1 = _threefry2x32(0, 42, np.zeros(size, np.uint32), cnt)
    bits = h0 ^ h1
    fl = ((bits >> np.uint32(9)) | np.uint32(0x3F800000)).view(np.float32)
    u = np.maximum(np.float32(0.0), fl - np.float32(1.0))
    return u.reshape(N, N)


def _get_order():
    global _ORDER_NP
    if _ORDER_NP is None:
        scores = _uniform_scores()
        np.fill_diagonal(scores, np.inf)
        _ORDER_NP = np.argsort(-scores, axis=1, kind="stable").astype(np.int32)
    return _ORDER_NP


def _adj_body(edges_hbm, zero_hbm, a_hbm, ebuf, arow):
    c = lax.axis_index("c")
    s = lax.axis_index("s")
    wid = s * NC + c
    base = wid * RW

    pltpu.sync_copy(zero_hbm, arow)
    pltpu.sync_copy(edges_hbm, ebuf)

    onesf = jnp.full((16,), 1.0, jnp.float32)

    def ebody(i, carry):
        aa = ebuf[pl.ds(i * 16, 16)]
        bb = ebuf[pl.ds(E + i * 16, 16)]
        ra = aa - base
        ma = (ra >= 0) & (ra < RW)
        ra_c = jnp.where(ma, ra, 0)
        plsc.store_scatter(arow, [ra_c * N + bb], onesf, mask=ma)
        rb = bb - base
        mb = (rb >= 0) & (rb < RW)
        rb_c = jnp.where(mb, rb, 0)
        plsc.store_scatter(arow, [rb_c * N + aa], onesf, mask=mb)
        return carry

    lax.fori_loop(0, E // 16, ebody, 0)

    iota16 = lax.iota(jnp.int32, 16)
    for ch in range(RW // 16):
        rv = iota16 + ch * 16
        plsc.store_scatter(arow, [rv * N + base + rv], onesf)

    pltpu.sync_copy(arow, a_hbm.at[pl.ds(base * N, RW * N)])


def _build_adj(edge_index):
    mesh = plsc.VectorSubcoreMesh(core_axis_name="c", subcore_axis_name="s")
    zero = jnp.zeros((RW * N,), jnp.float32)
    kfn = functools.partial(
        pl.kernel,
        mesh=mesh,
        compiler_params=pltpu.CompilerParams(needs_layout_passes=False),
        out_type=jax.ShapeDtypeStruct((N * N,), jnp.float32),
        scratch_types=[
            pltpu.VMEM((2 * E,), jnp.int32),
            pltpu.VMEM((RW * N,), jnp.float32),
        ],
    )(_adj_body)
    return kfn(edge_index.reshape(2 * E), zero).reshape(N, N)


def _mask_body(a_ref, m_ref):
    ab = a_ref[...].astype(jnp.bfloat16)
    cmat = lax.dot_general(ab, ab, (((1,), (0,)), ((), ())),
                           preferred_element_type=jnp.float32)
    m_ref[...] = (cmat > 0.0).astype(jnp.float32)


def _mask_tc(a):
    return pl.pallas_call(
        _mask_body,
        out_shape=jax.ShapeDtypeStruct((N, N), jnp.float32),
    )(a)


def _select_body(order_hbm, mask_hbm, h_hbm, hl_hbm, scnt_hbm,
                 morow, orow, idxb, scntv, grow, sem):
    c = lax.axis_index("c")
    s = lax.axis_index("s")
    wid = s * NC + c
    base = wid * RW

    pltpu.sync_copy(mask_hbm.at[pl.ds(base * N, RW * N)], morow)
    pltpu.sync_copy(order_hbm.at[pl.ds(base * N, RW * N)], orow)

    iota16 = lax.iota(jnp.int32, 16)
    lane0 = iota16 == 0

    def row_body(r, carry):
        selfv = jnp.broadcast_to(base + r, (16,)).astype(jnp.int32)
        idxb[pl.ds(r * M, 16)] = selfv
        idxb[pl.ds(r * M + 16, 16)] = selfv

        def chunk_body(k, found):
            ordv = orow[pl.ds(r * N + k * 16, 16)]
            mp = plsc.load_gather(morow, [r * N + ordv])
            mb = mp > 0.0
            cs = plsc.cumsum(mb.astype(jnp.int32))
            sel = mb & ((found + cs) <= MAXN)
            plsc.store_compressed(idxb.at[pl.ds(r * M + found, 16)], ordv,
                                  mask=sel)
            return found + jnp.sum(sel.astype(jnp.int32))

        found = lax.fori_loop(0, N // 16, chunk_body, jnp.int32(0))
        rsp = jnp.broadcast_to(r, (16,)).astype(jnp.int32)
        fsp = jnp.broadcast_to(found, (16,)).astype(jnp.int32)
        plsc.store_scatter(scntv, [rsp], fsp, mask=lane0)
        return carry

    lax.fori_loop(0, RW, row_body, 0)
    pltpu.sync_copy(scntv, scnt_hbm.at[pl.ds(base, RW)])

    def gath_body(r, carry):
        pltpu.async_copy(h_hbm.at[idxb.at[pl.ds(r * M, M)]], grow, sem).wait()
        pltpu.sync_copy(grow, hl_hbm.at[base + r])
        return carry

    lax.fori_loop(0, RW, gath_body, 0)


def _select_gather(order, maskf, H):
    mesh = plsc.VectorSubcoreMesh(core_axis_name="c", subcore_axis_name="s")
    kfn = functools.partial(
        pl.kernel,
        mesh=mesh,
        compiler_params=pltpu.CompilerParams(needs_layout_passes=False),
        out_type=(
            jax.ShapeDtypeStruct((N, M, D), jnp.float32),
            jax.ShapeDtypeStruct((N,), jnp.int32),
        ),
        scratch_types=[
            pltpu.VMEM((RW * N,), jnp.float32),
            pltpu.VMEM((RW * N,), jnp.int32),
            pltpu.VMEM((RW * M + 16,), jnp.int32),
            pltpu.VMEM((RW,), jnp.int32),
            pltpu.VMEM((M, D), jnp.float32),
            pltpu.SemaphoreType.DMA,
        ],
    )(_select_body)
    return kfn(order.reshape(N * N), maskf.reshape(N * N), H)


_BB = 32


def _bcast(x, shape, dims):
    return lax.broadcast_in_dim(x, shape, dims)


def _stats_body(hl_ref, scnt_ref, out_ref):
    G = hl_ref[...]
    scnt = scnt_ref[0, 0, :]
    sq = jnp.sum(G * G, axis=-1)
    inner = lax.dot_general(G, G, (((2,), (2,)), ((0,), (0,))),
                            preferred_element_type=jnp.float32)
    d2 = (_bcast(sq, (_BB, M, M), (0, 1)) + _bcast(sq, (_BB, M, M), (0, 2))
          - 2.0 * inner)
    d2 = jnp.maximum(d2, 0.0)
    dist = jnp.sqrt(d2 + 1e-12)

    mi = lax.broadcasted_iota(jnp.int32, (_BB, M, M), 1)
    ki = lax.broadcasted_iota(jnp.int32, (_BB, M, M), 2)
    sc3 = _bcast(scnt, (_BB, M, M), (0,))
    off = (mi < sc3) & (ki < sc3) & (mi != ki)
    dist_m = jnp.where(off, dist, BIG)
    pdrow = jnp.sum(jnp.where(off, dist, 0.0), axis=-1)

    keyi = lax.bitcast_convert_type(dist_m, jnp.int32)
    keyi = jnp.bitwise_or(jnp.bitwise_and(keyi, ~31), ki)
    cur = lax.bitcast_convert_type(keyi, jnp.float32)

    iota_m = lax.broadcasted_iota(jnp.int32, (_BB, M), 1)
    scb = _bcast(scnt, (_BB, M), (0,))
    valid2 = iota_m < scb

    zero = jnp.zeros((_BB, M), jnp.float32)
    s = zero
    s2 = zero
    xl = zero
    mx = zero
    nn = zero
    for t in range(KNEI):
        mn = jnp.min(cur, axis=-1)
        cur = jnp.where(cur == _bcast(mn, (_BB, M, M), (0, 1)), 1e38, cur)
        xv = lax.bitcast_convert_type(
            jnp.bitwise_and(lax.bitcast_convert_type(mn, jnp.int32), ~31),
            jnp.float32)
        wt = valid2 & ((t + 1) < scb)
        x = jnp.where(wt, xv, 0.0)
        s = s + x
        s2 = s2 + x * x
        xl = xl + x * jnp.log(x + 1e-12)
        mx = jnp.maximum(mx, x)
        nn = nn + wt.astype(jnp.float32)

    sum_knn = jnp.sum(s, axis=-1)
    n_raw = jnp.sum(nn, axis=-1)
    s2t = jnp.sum(s2, axis=-1)
    xlt = jnp.sum(xl, axis=-1)
    max_knn = jnp.max(mx, axis=-1)
    pd_t = jnp.sum(pdrow, axis=-1)

    n_k = n_raw + 1e-12
    mean_knn = sum_knn / n_k
    var = (s2t - 2.0 * mean_knn * sum_knn + mean_knn * mean_knn * n_raw) / n_k
    std_knn = jnp.sqrt(jnp.maximum(var, 0.0) + 1e-12)

    scf = scnt.astype(jnp.float32)
    n_pd = scf * scf - scf + 1e-12
    mean_pd = pd_t / n_pd

    sp = sum_knn + 1e-12
    ent = (sum_knn * jnp.log(sp) - xlt) / sp

    sigs = jnp.stack([mean_knn, max_knn, sum_knn, std_knn, mean_pd, ent],
                     axis=-1)
    gf = (scnt >= 3).astype(jnp.float32)
    sigs = sigs * _bcast(gf, (_BB, 6), (0,))
    out_ref[...] = jnp.concatenate(
        [sigs, jnp.zeros((_BB, 2), jnp.float32)], axis=-1)


def _stats_tc(hl, scnt3):
    return pl.pallas_call(
        _stats_body,
        grid=(N // _BB,),
        in_specs=[
            pl.BlockSpec((_BB, M, D), lambda i: (i, 0, 0)),
            pl.BlockSpec((1, 1, _BB), lambda i: (i, 0, 0)),
        ],
        out_specs=pl.BlockSpec((_BB, 8), lambda i: (i, 0)),
        out_shape=jax.ShapeDtypeStruct((N, 8), jnp.float32),
    )(hl, scnt3)


def kernel(H, edge_index):
    order = _get_order()
    a = _build_adj(edge_index)
    maskf = _mask_tc(a)
    hl, scnt = _select_gather(jnp.asarray(order), maskf, H)
    sigs8 = _stats_tc(hl, scnt.reshape(N // _BB, 1, _BB))
    return sigs8[:, :6]


# speedup vs baseline: 8.7034x; 1.5474x over previous
import functools

import numpy as np
import jax
import jax.numpy as jnp
from jax import lax
from jax.experimental import pallas as pl
from jax.experimental.pallas import tpu as pltpu
from jax.experimental.pallas import tpu_sc as plsc

N = 1024
E = 16384
D = 128
M = 32
MAXN = 30
KNEI = 5
BIG = 1e9

NC = 2
NS = 16
NW = NC * NS
RW = N // NW

_ORDER_NP = None


def _threefry2x32(k0, k1, x0, x1):
    def rotl(x, r):
        return ((x << np.uint32(r)) | (x >> np.uint32(32 - r))).astype(np.uint32)

    ks = [np.uint32(k0), np.uint32(k1),
          np.uint32(k0 ^ k1 ^ np.uint32(0x1BD11BDA))]
    rotations = [[13, 15, 26, 6], [17, 29, 16, 24]]
    x0 = (x0 + ks[0]).astype(np.uint32)
    x1 = (x1 + ks[1]).astype(np.uint32)
    for i in range(5):
        for r in rotations[i % 2]:
            x0 = (x0 + x1).astype(np.uint32)
            x1 = rotl(x1, r)
            x1 = x1 ^ x0
        x0 = (x0 + ks[(i + 1) % 3]).astype(np.uint32)
        x1 = (x1 + ks[(i + 2) % 3] + np.uint32(i + 1)).astype(np.uint32)
    return x0, x1


def _uniform_scores():
    size = N * N
    with np.errstate(over="ignore"):
        cnt = np.arange(size, dtype=np.uint32)
        h0, h1 = _threefry2x32(0, 42, np.zeros(size, np.uint32), cnt)
    bits = h0 ^ h1
    fl = ((bits >> np.uint32(9)) | np.uint32(0x3F800000)).view(np.float32)
    u = np.maximum(np.float32(0.0), fl - np.float32(1.0))
    return u.reshape(N, N)


def _get_order():
    global _ORDER_NP
    if _ORDER_NP is None:
        scores = _uniform_scores()
        np.fill_diagonal(scores, np.inf)
        _ORDER_NP = np.argsort(-scores, axis=1, kind="stable").astype(np.int32)
    return _ORDER_NP


def _adj_body(edges_hbm, zero_hbm, a_hbm, ebuf, arow):
    c = lax.axis_index("c")
    s = lax.axis_index("s")
    wid = s * NC + c
    base = wid * RW

    pltpu.sync_copy(zero_hbm, arow)
    pltpu.sync_copy(edges_hbm, ebuf)

    onesf = jnp.full((16,), 1.0, jnp.float32)

    def ebody(i, carry):
        aa = ebuf[pl.ds(i * 16, 16)]
        bb = ebuf[pl.ds(E + i * 16, 16)]
        ra = aa - base
        ma = (ra >= 0) & (ra < RW)
        ra_c = jnp.where(ma, ra, 0)
        plsc.store_scatter(arow, [ra_c * N + bb], onesf, mask=ma)
        rb = bb - base
        mb = (rb >= 0) & (rb < RW)
        rb_c = jnp.where(mb, rb, 0)
        plsc.store_scatter(arow, [rb_c * N + aa], onesf, mask=mb)
        return carry

    lax.fori_loop(0, E // 16, ebody, 0)

    iota16 = lax.iota(jnp.int32, 16)
    for ch in range(RW // 16):
        rv = iota16 + ch * 16
        plsc.store_scatter(arow, [rv * N + base + rv], onesf)

    pltpu.sync_copy(arow, a_hbm.at[pl.ds(base * N, RW * N)])


def _build_adj(edge_index):
    mesh = plsc.VectorSubcoreMesh(core_axis_name="c", subcore_axis_name="s")
    zero = jnp.zeros((RW * N,), jnp.float32)
    kfn = functools.partial(
        pl.kernel,
        mesh=mesh,
        compiler_params=pltpu.CompilerParams(needs_layout_passes=False),
        out_type=jax.ShapeDtypeStruct((N * N,), jnp.float32),
        scratch_types=[
            pltpu.VMEM((2 * E,), jnp.int32),
            pltpu.VMEM((RW * N,), jnp.float32),
        ],
    )(_adj_body)
    return kfn(edge_index.reshape(2 * E), zero).reshape(N, N)


def _mask_body(a_ref, m_ref):
    ab = a_ref[...].astype(jnp.bfloat16)
    cmat = lax.dot_general(ab, ab, (((1,), (0,)), ((), ())),
                           preferred_element_type=jnp.float32)
    m_ref[...] = (cmat > 0.0).astype(jnp.float32)


def _mask_tc(a):
    return pl.pallas_call(
        _mask_body,
        out_shape=jax.ShapeDtypeStruct((N, N), jnp.float32),
    )(a)


_GB = 8


def _select_body(order_hbm, mask_hbm, h_hbm, hl_hbm, scnt_hbm,
                 morow, orow, idxb, scntv, grow, sem):
    c = lax.axis_index("c")
    s = lax.axis_index("s")
    wid = s * NC + c
    base = wid * RW

    pltpu.sync_copy(mask_hbm.at[pl.ds(base * N, RW * N)], morow)
    pltpu.sync_copy(order_hbm.at[pl.ds(base * N, RW * N)], orow)

    iota16 = lax.iota(jnp.int32, 16)
    lane0 = iota16 == 0

    def row_body(r, carry):
        selfv = jnp.broadcast_to(base + r, (16,)).astype(jnp.int32)
        idxb[pl.ds(r * M, 16)] = selfv
        idxb[pl.ds(r * M + 16, 16)] = selfv

        def chunk_cond(carry):
            k, found = carry
            return (k < N // 16) & (found < MAXN)

        def chunk_body(carry):
            k, found = carry
            ordv = orow[pl.ds(r * N + k * 16, 16)]
            mp = plsc.load_gather(morow, [r * N + ordv])
            mb = mp > 0.0
            cs = plsc.cumsum(mb.astype(jnp.int32))
            sel = mb & ((found + cs) <= MAXN)
            plsc.store_compressed(idxb.at[pl.ds(r * M + found, 16)], ordv,
                                  mask=sel)
            return k + 1, found + jnp.sum(sel.astype(jnp.int32))

        _, found = lax.while_loop(chunk_cond, chunk_body,
                                  (jnp.int32(0), jnp.int32(0)))
        rsp = jnp.broadcast_to(r, (16,)).astype(jnp.int32)
        fsp = jnp.broadcast_to(found, (16,)).astype(jnp.int32)
        plsc.store_scatter(scntv, [rsp], fsp, mask=lane0)
        return carry

    lax.fori_loop(0, RW, row_body, 0)
    pltpu.sync_copy(scntv, scnt_hbm.at[pl.ds(base, RW)])

    def gath_body(b, carry):
        pltpu.async_copy(h_hbm.at[idxb.at[pl.ds(b * _GB * M, _GB * M)]],
                         grow, sem).wait()
        pltpu.sync_copy(grow, hl_hbm.at[pl.ds((base + b * _GB) * M, _GB * M)])
        return carry

    lax.fori_loop(0, RW // _GB, gath_body, 0)


def _select_gather(order, maskf, H):
    mesh = plsc.VectorSubcoreMesh(core_axis_name="c", subcore_axis_name="s")
    kfn = functools.partial(
        pl.kernel,
        mesh=mesh,
        compiler_params=pltpu.CompilerParams(needs_layout_passes=False),
        out_type=(
            jax.ShapeDtypeStruct((N * M, D), jnp.float32),
            jax.ShapeDtypeStruct((N,), jnp.int32),
        ),
        scratch_types=[
            pltpu.VMEM((RW * N,), jnp.float32),
            pltpu.VMEM((RW * N,), jnp.int32),
            pltpu.VMEM((RW * M + 16,), jnp.int32),
            pltpu.VMEM((RW,), jnp.int32),
            pltpu.VMEM((_GB * M, D), jnp.float32),
            pltpu.SemaphoreType.DMA,
        ],
    )(_select_body)
    hl, scnt = kfn(order.reshape(N * N), maskf.reshape(N * N), H)
    return hl.reshape(N, M, D), scnt


_BB = 32


def _bcast(x, shape, dims):
    return lax.broadcast_in_dim(x, shape, dims)


def _stats_body(hl_ref, scnt_ref, out_ref):
    G = hl_ref[...]
    scnt = scnt_ref[0, 0, :]
    sq = jnp.sum(G * G, axis=-1)
    inner = lax.dot_general(G, G, (((2,), (2,)), ((0,), (0,))),
                            preferred_element_type=jnp.float32)
    d2 = (_bcast(sq, (_BB, M, M), (0, 1)) + _bcast(sq, (_BB, M, M), (0, 2))
          - 2.0 * inner)
    d2 = jnp.maximum(d2, 0.0)
    dist = jnp.sqrt(d2 + 1e-12)

    mi = lax.broadcasted_iota(jnp.int32, (_BB, M, M), 1)
    ki = lax.broadcasted_iota(jnp.int32, (_BB, M, M), 2)
    sc3 = _bcast(scnt, (_BB, M, M), (0,))
    off = (mi < sc3) & (ki < sc3) & (mi != ki)
    dist_m = jnp.where(off, dist, BIG)
    pdrow = jnp.sum(jnp.where(off, dist, 0.0), axis=-1)

    keyi = lax.bitcast_convert_type(dist_m, jnp.int32)
    keyi = jnp.bitwise_or(jnp.bitwise_and(keyi, ~31), ki)
    cur = lax.bitcast_convert_type(keyi, jnp.float32)

    iota_m = lax.broadcasted_iota(jnp.int32, (_BB, M), 1)
    scb = _bcast(scnt, (_BB, M), (0,))
    valid2 = iota_m < scb

    zero = jnp.zeros((_BB, M), jnp.float32)
    s = zero
    s2 = zero
    xl = zero
    mx = zero
    nn = zero
    for t in range(KNEI):
        mn = jnp.min(cur, axis=-1)
        cur = jnp.where(cur == _bcast(mn, (_BB, M, M), (0, 1)), 1e38, cur)
        xv = lax.bitcast_convert_type(
            jnp.bitwise_and(lax.bitcast_convert_type(mn, jnp.int32), ~31),
            jnp.float32)
        wt = valid2 & ((t + 1) < scb)
        x = jnp.where(wt, xv, 0.0)
        s = s + x
        s2 = s2 + x * x
        xl = xl + x * jnp.log(x + 1e-12)
        mx = jnp.maximum(mx, x)
        nn = nn + wt.astype(jnp.float32)

    sum_knn = jnp.sum(s, axis=-1)
    n_raw = jnp.sum(nn, axis=-1)
    s2t = jnp.sum(s2, axis=-1)
    xlt = jnp.sum(xl, axis=-1)
    max_knn = jnp.max(mx, axis=-1)
    pd_t = jnp.sum(pdrow, axis=-1)

    n_k = n_raw + 1e-12
    mean_knn = sum_knn / n_k
    var = (s2t - 2.0 * mean_knn * sum_knn + mean_knn * mean_knn * n_raw) / n_k
    std_knn = jnp.sqrt(jnp.maximum(var, 0.0) + 1e-12)

    scf = scnt.astype(jnp.float32)
    n_pd = scf * scf - scf + 1e-12
    mean_pd = pd_t / n_pd

    sp = sum_knn + 1e-12
    ent = (sum_knn * jnp.log(sp) - xlt) / sp

    sigs = jnp.stack([mean_knn, max_knn, sum_knn, std_knn, mean_pd, ent],
                     axis=-1)
    gf = (scnt >= 3).astype(jnp.float32)
    sigs = sigs * _bcast(gf, (_BB, 6), (0,))
    out_ref[...] = jnp.concatenate(
        [sigs, jnp.zeros((_BB, 2), jnp.float32)], axis=-1)


def _stats_tc(hl, scnt3):
    return pl.pallas_call(
        _stats_body,
        grid=(N // _BB,),
        in_specs=[
            pl.BlockSpec((_BB, M, D), lambda i: (i, 0, 0)),
            pl.BlockSpec((1, 1, _BB), lambda i: (i, 0, 0)),
        ],
        out_specs=pl.BlockSpec((_BB, 8), lambda i: (i, 0)),
        out_shape=jax.ShapeDtypeStruct((N, 8), jnp.float32),
    )(hl, scnt3)


def kernel(H, edge_index):
    order = _get_order()
    a = _build_adj(edge_index)
    maskf = _mask_tc(a)
    hl, scnt = _select_gather(jnp.asarray(order), maskf, H)
    sigs8 = _stats_tc(hl, scnt.reshape(N // _BB, 1, _BB))
    return sigs8[:, :6]


# speedup vs baseline: 8.9111x; 1.0239x over previous
import functools

import numpy as np
import jax
import jax.numpy as jnp
from jax import lax
from jax.experimental import pallas as pl
from jax.experimental.pallas import tpu as pltpu
from jax.experimental.pallas import tpu_sc as plsc

N = 1024
E = 16384
D = 128
M = 32
MAXN = 30
KNEI = 5
BIG = 1e9

NC = 2
NS = 16
NW = NC * NS
RW = N // NW

_ORDER_NP = None


def _threefry2x32(k0, k1, x0, x1):
    def rotl(x, r):
        return ((x << np.uint32(r)) | (x >> np.uint32(32 - r))).astype(np.uint32)

    ks = [np.uint32(k0), np.uint32(k1),
          np.uint32(k0 ^ k1 ^ np.uint32(0x1BD11BDA))]
    rotations = [[13, 15, 26, 6], [17, 29, 16, 24]]
    x0 = (x0 + ks[0]).astype(np.uint32)
    x1 = (x1 + ks[1]).astype(np.uint32)
    for i in range(5):
        for r in rotations[i % 2]:
            x0 = (x0 + x1).astype(np.uint32)
            x1 = rotl(x1, r)
            x1 = x1 ^ x0
        x0 = (x0 + ks[(i + 1) % 3]).astype(np.uint32)
        x1 = (x1 + ks[(i + 2) % 3] + np.uint32(i + 1)).astype(np.uint32)
    return x0, x1


def _uniform_scores():
    size = N * N
    with np.errstate(over="ignore"):
        cnt = np.arange(size, dtype=np.uint32)
        h0, h1 = _threefry2x32(0, 42, np.zeros(size, np.uint32), cnt)
    bits = h0 ^ h1
    fl = ((bits >> np.uint32(9)) | np.uint32(0x3F800000)).view(np.float32)
    u = np.maximum(np.float32(0.0), fl - np.float32(1.0))
    return u.reshape(N, N)


def _get_order():
    global _ORDER_NP
    if _ORDER_NP is None:
        scores = _uniform_scores()
        np.fill_diagonal(scores, np.inf)
        _ORDER_NP = np.argsort(-scores, axis=1, kind="stable").astype(np.int32)
    return _ORDER_NP


def _adj_body(edges_hbm, zero_hbm, a_hbm, ebuf, arow):
    c = lax.axis_index("c")
    s = lax.axis_index("s")
    wid = s * NC + c
    base = wid * RW

    pltpu.sync_copy(zero_hbm, arow)
    pltpu.sync_copy(edges_hbm, ebuf)

    onesf = jnp.full((16,), 1.0, jnp.float32)

    @plsc.parallel_loop(0, E // 16, step=1, unroll=4)
    def ebody(i):
        aa = ebuf[pl.ds(i * 16, 16)]
        bb = ebuf[pl.ds(E + i * 16, 16)]
        ra = aa - base
        ma = (ra >= 0) & (ra < RW)
        ra_c = jnp.where(ma, ra, 0)
        plsc.store_scatter(arow, [ra_c * N + bb], onesf, mask=ma)
        rb = bb - base
        mb = (rb >= 0) & (rb < RW)
        rb_c = jnp.where(mb, rb, 0)
        plsc.store_scatter(arow, [rb_c * N + aa], onesf, mask=mb)

    iota16 = lax.iota(jnp.int32, 16)
    for ch in range(RW // 16):
        rv = iota16 + ch * 16
        plsc.store_scatter(arow, [rv * N + base + rv], onesf)

    pltpu.sync_copy(arow, a_hbm.at[pl.ds(base * N, RW * N)])


def _build_adj(edge_index):
    mesh = plsc.VectorSubcoreMesh(core_axis_name="c", subcore_axis_name="s")
    zero = jnp.zeros((RW * N,), jnp.float32)
    kfn = functools.partial(
        pl.kernel,
        mesh=mesh,
        compiler_params=pltpu.CompilerParams(needs_layout_passes=False),
        out_type=jax.ShapeDtypeStruct((N * N,), jnp.float32),
        scratch_types=[
            pltpu.VMEM((2 * E,), jnp.int32),
            pltpu.VMEM((RW * N,), jnp.float32),
        ],
    )(_adj_body)
    return kfn(edge_index.reshape(2 * E), zero).reshape(N, N)


def _mask_body(a_ref, m_ref):
    ab = a_ref[...].astype(jnp.bfloat16)
    cmat = lax.dot_general(ab, ab, (((1,), (0,)), ((), ())),
                           preferred_element_type=jnp.float32)
    m_ref[...] = (cmat > 0.0).astype(jnp.float32)


def _mask_tc(a):
    return pl.pallas_call(
        _mask_body,
        out_shape=jax.ShapeDtypeStruct((N, N), jnp.float32),
    )(a)


_GB = 8


def _select_body(order_hbm, mask_hbm, h_hbm, hl_hbm, scnt_hbm,
                 morow, orow, idxb, scntv, grow, gsem):
    c = lax.axis_index("c")
    s = lax.axis_index("s")
    wid = s * NC + c
    base = wid * RW

    pltpu.sync_copy(mask_hbm.at[pl.ds(base * N, RW * N)], morow)
    pltpu.sync_copy(order_hbm.at[pl.ds(base * N, RW * N)], orow)

    iota16 = lax.iota(jnp.int32, 16)
    lane0 = iota16 == 0

    def row_body(r, carry):
        selfv = jnp.broadcast_to(base + r, (16,)).astype(jnp.int32)
        idxb[pl.ds(r * M, 16)] = selfv
        idxb[pl.ds(r * M + 16, 16)] = selfv

        def chunk_cond(carry):
            k, found = carry
            return (k < N // 16) & (found < MAXN)

        def chunk_body(carry):
            k, found = carry
            ordv = orow[pl.ds(r * N + k * 16, 16)]
            mp = plsc.load_gather(morow, [r * N + ordv])
            mb = mp > 0.0
            cs = plsc.cumsum(mb.astype(jnp.int32))
            sel = mb & ((found + cs) <= MAXN)
            plsc.store_compressed(idxb.at[pl.ds(r * M + found, 16)], ordv,
                                  mask=sel)
            return k + 1, found + jnp.sum(sel.astype(jnp.int32))

        _, found = lax.while_loop(chunk_cond, chunk_body,
                                  (jnp.int32(0), jnp.int32(0)))
        rsp = jnp.broadcast_to(r, (16,)).astype(jnp.int32)
        fsp = jnp.broadcast_to(found, (16,)).astype(jnp.int32)
        plsc.store_scatter(scntv, [rsp], fsp, mask=lane0)
        return carry

    nb = RW // _GB
    gd = [None] * nb
    for b in range(nb):
        lax.fori_loop(b * _GB, (b + 1) * _GB, row_body, 0)
        if b >= 1:
            gd[b - 1].wait()
            pltpu.sync_copy(
                grow, hl_hbm.at[pl.ds((base + (b - 1) * _GB) * M, _GB * M)])
        gd[b] = pltpu.async_copy(
            h_hbm.at[idxb.at[pl.ds(b * _GB * M, _GB * M)]], grow, gsem)
    gd[nb - 1].wait()
    pltpu.sync_copy(
        grow, hl_hbm.at[pl.ds((base + (nb - 1) * _GB) * M, _GB * M)])

    pltpu.sync_copy(scntv, scnt_hbm.at[pl.ds(base, RW)])


def _select_gather(order, maskf, H):
    mesh = plsc.VectorSubcoreMesh(core_axis_name="c", subcore_axis_name="s")
    kfn = functools.partial(
        pl.kernel,
        mesh=mesh,
        compiler_params=pltpu.CompilerParams(needs_layout_passes=False),
        out_type=(
            jax.ShapeDtypeStruct((N * M, D), jnp.float32),
            jax.ShapeDtypeStruct((N,), jnp.int32),
        ),
        scratch_types=[
            pltpu.VMEM((RW * N,), jnp.float32),
            pltpu.VMEM((RW * N,), jnp.int32),
            pltpu.VMEM((RW * M + 16,), jnp.int32),
            pltpu.VMEM((RW,), jnp.int32),
            pltpu.VMEM((_GB * M, D), jnp.float32),
            pltpu.SemaphoreType.DMA,
        ],
    )(_select_body)
    hl, scnt = kfn(order.reshape(N * N), maskf.reshape(N * N), H)
    return hl.reshape(N, M, D), scnt


_BB = 64


def _bcast(x, shape, dims):
    return lax.broadcast_in_dim(x, shape, dims)


def _stats_body(hl_ref, scnt_ref, out_ref):
    G = hl_ref[...]
    scnt = scnt_ref[0, 0, :]
    sq = jnp.sum(G * G, axis=-1)
    inner = lax.dot_general(G, G, (((2,), (2,)), ((0,), (0,))),
                            preferred_element_type=jnp.float32)
    d2 = (_bcast(sq, (_BB, M, M), (0, 1)) + _bcast(sq, (_BB, M, M), (0, 2))
          - 2.0 * inner)
    d2 = jnp.maximum(d2, 0.0)
    dist = jnp.sqrt(d2 + 1e-12)

    mi = lax.broadcasted_iota(jnp.int32, (_BB, M, M), 1)
    ki = lax.broadcasted_iota(jnp.int32, (_BB, M, M), 2)
    sc3 = _bcast(scnt, (_BB, M, M), (0,))
    off = (mi < sc3) & (ki < sc3) & (mi != ki)
    dist_m = jnp.where(off, dist, BIG)
    pdrow = jnp.sum(jnp.where(off, dist, 0.0), axis=-1)

    keyi = lax.bitcast_convert_type(dist_m, jnp.int32)
    keyi = jnp.bitwise_or(jnp.bitwise_and(keyi, ~31), ki)
    cur = lax.bitcast_convert_type(keyi, jnp.float32)

    iota_m = lax.broadcasted_iota(jnp.int32, (_BB, M), 1)
    scb = _bcast(scnt, (_BB, M), (0,))
    valid2 = iota_m < scb

    zero = jnp.zeros((_BB, M), jnp.float32)
    s = zero
    s2 = zero
    xl = zero
    mx = zero
    nn = zero
    for t in range(KNEI):
        mn = jnp.min(cur, axis=-1)
        cur = jnp.where(cur == _bcast(mn, (_BB, M, M), (0, 1)), 1e38, cur)
        xv = lax.bitcast_convert_type(
            jnp.bitwise_and(lax.bitcast_convert_type(mn, jnp.int32), ~31),
            jnp.float32)
        wt = valid2 & ((t + 1) < scb)
        x = jnp.where(wt, xv, 0.0)
        s = s + x
        s2 = s2 + x * x
        xl = xl + x * jnp.log(x + 1e-12)
        mx = jnp.maximum(mx, x)
        nn = nn + wt.astype(jnp.float32)

    sum_knn = jnp.sum(s, axis=-1)
    n_raw = jnp.sum(nn, axis=-1)
    s2t = jnp.sum(s2, axis=-1)
    xlt = jnp.sum(xl, axis=-1)
    max_knn = jnp.max(mx, axis=-1)
    pd_t = jnp.sum(pdrow, axis=-1)

    n_k = n_raw + 1e-12
    mean_knn = sum_knn / n_k
    var = (s2t - 2.0 * mean_knn * sum_knn + mean_knn * mean_knn * n_raw) / n_k
    std_knn = jnp.sqrt(jnp.maximum(var, 0.0) + 1e-12)

    scf = scnt.astype(jnp.float32)
    n_pd = scf * scf - scf + 1e-12
    mean_pd = pd_t / n_pd

    sp = sum_knn + 1e-12
    ent = (sum_knn * jnp.log(sp) - xlt) / sp

    sigs = jnp.stack([mean_knn, max_knn, sum_knn, std_knn, mean_pd, ent],
                     axis=-1)
    gf = (scnt >= 3).astype(jnp.float32)
    sigs = sigs * _bcast(gf, (_BB, 6), (0,))
    out_ref[...] = jnp.concatenate(
        [sigs, jnp.zeros((_BB, 2), jnp.float32)], axis=-1)


def _stats_tc(hl, scnt3):
    return pl.pallas_call(
        _stats_body,
        grid=(N // _BB,),
        in_specs=[
            pl.BlockSpec((_BB, M, D), lambda i: (i, 0, 0)),
            pl.BlockSpec((1, 1, _BB), lambda i: (i, 0, 0)),
        ],
        out_specs=pl.BlockSpec((_BB, 8), lambda i: (i, 0)),
        out_shape=jax.ShapeDtypeStruct((N, 8), jnp.float32),
    )(hl, scnt3)


def kernel(H, edge_index):
    order = _get_order()
    a = _build_adj(edge_index)
    maskf = _mask_tc(a)
    hl, scnt = _select_gather(jnp.asarray(order), maskf, H)
    sigs8 = _stats_tc(hl, scnt.reshape(N // _BB, 1, _BB))
    return sigs8[:, :6]


# speedup vs baseline: 9.5271x; 1.0691x over previous
import functools

import numpy as np
import jax
import jax.numpy as jnp
from jax import lax
from jax.experimental import pallas as pl
from jax.experimental.pallas import tpu as pltpu
from jax.experimental.pallas import tpu_sc as plsc

N = 1024
E = 16384
D = 128
M = 32
MAXN = 30
KNEI = 5
BIG = 1e9

NC = 2
NS = 16
NW = NC * NS
RW = N // NW

_ORDER_NP = None


def _threefry2x32(k0, k1, x0, x1):
    def rotl(x, r):
        return ((x << np.uint32(r)) | (x >> np.uint32(32 - r))).astype(np.uint32)

    ks = [np.uint32(k0), np.uint32(k1),
          np.uint32(k0 ^ k1 ^ np.uint32(0x1BD11BDA))]
    rotations = [[13, 15, 26, 6], [17, 29, 16, 24]]
    x0 = (x0 + ks[0]).astype(np.uint32)
    x1 = (x1 + ks[1]).astype(np.uint32)
    for i in range(5):
        for r in rotations[i % 2]:
            x0 = (x0 + x1).astype(np.uint32)
            x1 = rotl(x1, r)
            x1 = x1 ^ x0
        x0 = (x0 + ks[(i + 1) % 3]).astype(np.uint32)
        x1 = (x1 + ks[(i + 2) % 3] + np.uint32(i + 1)).astype(np.uint32)
    return x0, x1


def _uniform_scores():
    size = N * N
    with np.errstate(over="ignore"):
        cnt = np.arange(size, dtype=np.uint32)
        h0, h1 = _threefry2x32(0, 42, np.zeros(size, np.uint32), cnt)
    bits = h0 ^ h1
    fl = ((bits >> np.uint32(9)) | np.uint32(0x3F800000)).view(np.float32)
    u = np.maximum(np.float32(0.0), fl - np.float32(1.0))
    return u.reshape(N, N)


def _get_order():
    global _ORDER_NP
    if _ORDER_NP is None:
        scores = _uniform_scores()
        np.fill_diagonal(scores, np.inf)
        _ORDER_NP = np.argsort(-scores, axis=1, kind="stable").astype(np.int32)
    return _ORDER_NP


def _adj_body(edges_hbm, zero_hbm, a_hbm, ebuf, arow):
    c = lax.axis_index("c")
    s = lax.axis_index("s")
    wid = s * NC + c
    base = wid * RW

    pltpu.sync_copy(zero_hbm, arow)
    pltpu.sync_copy(edges_hbm, ebuf)

    onesf = jnp.full((16,), 1.0, jnp.float32)

    @plsc.parallel_loop(0, E // 16, step=1, unroll=4)
    def ebody(i):
        aa = ebuf[pl.ds(i * 16, 16)]
        bb = ebuf[pl.ds(E + i * 16, 16)]
        ra = aa - base
        ma = (ra >= 0) & (ra < RW)
        ra_c = jnp.where(ma, ra, 0)
        plsc.store_scatter(arow, [ra_c, bb], onesf, mask=ma)
        rb = bb - base
        mb = (rb >= 0) & (rb < RW)
        rb_c = jnp.where(mb, rb, 0)
        plsc.store_scatter(arow, [rb_c, aa], onesf, mask=mb)

    iota16 = lax.iota(jnp.int32, 16)
    for ch in range(RW // 16):
        rv = iota16 + ch * 16
        plsc.store_scatter(arow, [rv, base + rv], onesf)

    pltpu.sync_copy(arow, a_hbm.at[pl.ds(base, RW)])


def _build_adj(edge_index):
    mesh = plsc.VectorSubcoreMesh(core_axis_name="c", subcore_axis_name="s")
    zero = jnp.zeros((RW, N), jnp.float32)
    kfn = functools.partial(
        pl.kernel,
        mesh=mesh,
        compiler_params=pltpu.CompilerParams(needs_layout_passes=False),
        out_type=jax.ShapeDtypeStruct((N, N), jnp.float32),
        scratch_types=[
            pltpu.VMEM((2 * E,), jnp.int32),
            pltpu.VMEM((RW, N), jnp.float32),
        ],
    )(_adj_body)
    return kfn(edge_index.reshape(2 * E), zero)


def _mask_body(a_ref, m_ref):
    ab = a_ref[...].astype(jnp.bfloat16)
    cmat = lax.dot_general(ab, ab, (((1,), (0,)), ((), ())),
                           preferred_element_type=jnp.float32)
    m_ref[...] = (cmat > 0.0).astype(jnp.float32)


def _mask_tc(a):
    return pl.pallas_call(
        _mask_body,
        out_shape=jax.ShapeDtypeStruct((N, N), jnp.float32),
    )(a)


_GB = 8


def _select_body(order_hbm, mask_hbm, h_hbm, hl_hbm, scnt_hbm,
                 morow, orow, idxb, scntv, grow, gsem):
    c = lax.axis_index("c")
    s = lax.axis_index("s")
    wid = s * NC + c
    base = wid * RW

    pltpu.sync_copy(mask_hbm.at[pl.ds(base, RW)], morow)
    pltpu.sync_copy(order_hbm.at[pl.ds(base * N, RW * N)], orow)

    iota16 = lax.iota(jnp.int32, 16)
    lane0 = iota16 == 0

    def row_body(r, carry):
        selfv = jnp.broadcast_to(base + r, (16,)).astype(jnp.int32)
        idxb[pl.ds(r * M, 16)] = selfv
        idxb[pl.ds(r * M + 16, 16)] = selfv

        def chunk_cond(carry):
            k, found = carry
            return (k < N // 16) & (found < MAXN)

        def chunk_body(carry):
            k, found = carry
            ordv = orow[pl.ds(r * N + k * 16, 16)]
            rsp = jnp.broadcast_to(r, (16,)).astype(jnp.int32)
            mp = plsc.load_gather(morow, [rsp, ordv])
            mb = mp > 0.0
            cs = plsc.cumsum(mb.astype(jnp.int32))
            sel = mb & ((found + cs) <= MAXN)
            plsc.store_compressed(idxb.at[pl.ds(r * M + found, 16)], ordv,
                                  mask=sel)
            return k + 1, found + jnp.sum(sel.astype(jnp.int32))

        _, found = lax.while_loop(chunk_cond, chunk_body,
                                  (jnp.int32(0), jnp.int32(0)))
        rsp = jnp.broadcast_to(r, (16,)).astype(jnp.int32)
        fsp = jnp.broadcast_to(found, (16,)).astype(jnp.int32)
        plsc.store_scatter(scntv, [rsp], fsp, mask=lane0)
        return carry

    nb = RW // _GB
    gd = [None] * nb
    for b in range(nb):
        lax.fori_loop(b * _GB, (b + 1) * _GB, row_body, 0)
        if b >= 1:
            gd[b - 1].wait()
            pltpu.sync_copy(
                grow, hl_hbm.at[pl.ds((base + (b - 1) * _GB) * M, _GB * M)])
        gd[b] = pltpu.async_copy(
            h_hbm.at[idxb.at[pl.ds(b * _GB * M, _GB * M)]], grow, gsem)
    gd[nb - 1].wait()
    pltpu.sync_copy(
        grow, hl_hbm.at[pl.ds((base + (nb - 1) * _GB) * M, _GB * M)])

    pltpu.sync_copy(scntv, scnt_hbm.at[pl.ds(base, RW)])


def _select_gather(order, maskf, H):
    mesh = plsc.VectorSubcoreMesh(core_axis_name="c", subcore_axis_name="s")
    kfn = functools.partial(
        pl.kernel,
        mesh=mesh,
        compiler_params=pltpu.CompilerParams(needs_layout_passes=False),
        out_type=(
            jax.ShapeDtypeStruct((N * M, D), jnp.float32),
            jax.ShapeDtypeStruct((N,), jnp.int32),
        ),
        scratch_types=[
            pltpu.VMEM((RW, N), jnp.float32),
            pltpu.VMEM((RW * N,), jnp.int32),
            pltpu.VMEM((RW * M + 16,), jnp.int32),
            pltpu.VMEM((RW,), jnp.int32),
            pltpu.VMEM((_GB * M, D), jnp.float32),
            pltpu.SemaphoreType.DMA,
        ],
    )(_select_body)
    hl, scnt = kfn(order.reshape(N * N), maskf, H)
    return hl.reshape(N, M, D), scnt


_BB = 64


def _bcast(x, shape, dims):
    return lax.broadcast_in_dim(x, shape, dims)


def _stats_body(hl_ref, scnt_ref, out_ref):
    G = hl_ref[...]
    scnt = scnt_ref[0, 0, :]
    sq = jnp.sum(G * G, axis=-1)
    inner = lax.dot_general(G, G, (((2,), (2,)), ((0,), (0,))),
                            preferred_element_type=jnp.float32)
    d2 = (_bcast(sq, (_BB, M, M), (0, 1)) + _bcast(sq, (_BB, M, M), (0, 2))
          - 2.0 * inner)
    d2 = jnp.maximum(d2, 0.0)
    dist = jnp.sqrt(d2 + 1e-12)

    mi = lax.broadcasted_iota(jnp.int32, (_BB, M, M), 1)
    ki = lax.broadcasted_iota(jnp.int32, (_BB, M, M), 2)
    sc3 = _bcast(scnt, (_BB, M, M), (0,))
    off = (mi < sc3) & (ki < sc3) & (mi != ki)
    dist_m = jnp.where(off, dist, BIG)
    pdrow = jnp.sum(jnp.where(off, dist, 0.0), axis=-1)

    keyi = lax.bitcast_convert_type(dist_m, jnp.int32)
    keyi = jnp.bitwise_or(jnp.bitwise_and(keyi, ~31), ki)
    cur = lax.bitcast_convert_type(keyi, jnp.float32)

    iota_m = lax.broadcasted_iota(jnp.int32, (_BB, M), 1)
    scb = _bcast(scnt, (_BB, M), (0,))
    valid2 = iota_m < scb

    zero = jnp.zeros((_BB, M), jnp.float32)
    s = zero
    s2 = zero
    xl = zero
    mx = zero
    nn = zero
    for t in range(KNEI):
        mn = jnp.min(cur, axis=-1)
        cur = jnp.where(cur == _bcast(mn, (_BB, M, M), (0, 1)), 1e38, cur)
        xv = lax.bitcast_convert_type(
            jnp.bitwise_and(lax.bitcast_convert_type(mn, jnp.int32), ~31),
            jnp.float32)
        wt = valid2 & ((t + 1) < scb)
        x = jnp.where(wt, xv, 0.0)
        s = s + x
        s2 = s2 + x * x
        xl = xl + x * jnp.log(x + 1e-12)
        mx = jnp.maximum(mx, x)
        nn = nn + wt.astype(jnp.float32)

    sum_knn = jnp.sum(s, axis=-1)
    n_raw = jnp.sum(nn, axis=-1)
    s2t = jnp.sum(s2, axis=-1)
    xlt = jnp.sum(xl, axis=-1)
    max_knn = jnp.max(mx, axis=-1)
    pd_t = jnp.sum(pdrow, axis=-1)

    n_k = n_raw + 1e-12
    mean_knn = sum_knn / n_k
    var = (s2t - 2.0 * mean_knn * sum_knn + mean_knn * mean_knn * n_raw) / n_k
    std_knn = jnp.sqrt(jnp.maximum(var, 0.0) + 1e-12)

    scf = scnt.astype(jnp.float32)
    n_pd = scf * scf - scf + 1e-12
    mean_pd = pd_t / n_pd

    sp = sum_knn + 1e-12
    ent = (sum_knn * jnp.log(sp) - xlt) / sp

    sigs = jnp.stack([mean_knn, max_knn, sum_knn, std_knn, mean_pd, ent],
                     axis=-1)
    gf = (scnt >= 3).astype(jnp.float32)
    out_ref[...] = sigs * _bcast(gf, (_BB, 6), (0,))


def _stats_tc(hl, scnt3):
    return pl.pallas_call(
        _stats_body,
        grid=(N // _BB,),
        in_specs=[
            pl.BlockSpec((_BB, M, D), lambda i: (i, 0, 0)),
            pl.BlockSpec((1, 1, _BB), lambda i: (i, 0, 0)),
        ],
        out_specs=pl.BlockSpec((_BB, 6), lambda i: (i, 0)),
        out_shape=jax.ShapeDtypeStruct((N, 6), jnp.float32),
    )(hl, scnt3)


def kernel(H, edge_index):
    order = _get_order()
    a = _build_adj(edge_index)
    maskf = _mask_tc(a)
    hl, scnt = _select_gather(jnp.asarray(order), maskf, H)
    return _stats_tc(hl, scnt.reshape(N // _BB, 1, _BB))


# speedup vs baseline: 9.8935x; 1.0385x over previous
import functools

import numpy as np
import jax
import jax.numpy as jnp
from jax import lax
from jax.experimental import pallas as pl
from jax.experimental.pallas import tpu as pltpu
from jax.experimental.pallas import tpu_sc as plsc

N = 1024
E = 16384
D = 128
M = 32
MAXN = 30
KNEI = 5
BIG = 1e9

NC = 2
NS = 16
NW = NC * NS
RW = N // NW

_ORDER_NP = None


def _threefry2x32(k0, k1, x0, x1):
    def rotl(x, r):
        return ((x << np.uint32(r)) | (x >> np.uint32(32 - r))).astype(np.uint32)

    ks = [np.uint32(k0), np.uint32(k1),
          np.uint32(k0 ^ k1 ^ np.uint32(0x1BD11BDA))]
    rotations = [[13, 15, 26, 6], [17, 29, 16, 24]]
    x0 = (x0 + ks[0]).astype(np.uint32)
    x1 = (x1 + ks[1]).astype(np.uint32)
    for i in range(5):
        for r in rotations[i % 2]:
            x0 = (x0 + x1).astype(np.uint32)
            x1 = rotl(x1, r)
            x1 = x1 ^ x0
        x0 = (x0 + ks[(i + 1) % 3]).astype(np.uint32)
        x1 = (x1 + ks[(i + 2) % 3] + np.uint32(i + 1)).astype(np.uint32)
    return x0, x1


def _uniform_scores():
    size = N * N
    with np.errstate(over="ignore"):
        cnt = np.arange(size, dtype=np.uint32)
        h0, h1 = _threefry2x32(0, 42, np.zeros(size, np.uint32), cnt)
    bits = h0 ^ h1
    fl = ((bits >> np.uint32(9)) | np.uint32(0x3F800000)).view(np.float32)
    u = np.maximum(np.float32(0.0), fl - np.float32(1.0))
    return u.reshape(N, N)


def _get_order():
    global _ORDER_NP
    if _ORDER_NP is None:
        scores = _uniform_scores()
        np.fill_diagonal(scores, np.inf)
        _ORDER_NP = np.argsort(-scores, axis=1, kind="stable").astype(np.int32)
    return _ORDER_NP


def _adj_body(edges_hbm, zero_hbm, a_hbm, ebuf, arow):
    c = lax.axis_index("c")
    s = lax.axis_index("s")
    wid = s * NC + c
    base = wid * RW

    pltpu.sync_copy(zero_hbm, arow)
    pltpu.sync_copy(edges_hbm, ebuf)

    onesf = jnp.full((16,), 1.0, jnp.float32)

    @plsc.parallel_loop(0, E // 16, step=1, unroll=4)
    def ebody(i):
        aa = ebuf[pl.ds(i * 16, 16)]
        bb = ebuf[pl.ds(E + i * 16, 16)]
        ra = aa - base
        ma = (ra >= 0) & (ra < RW)
        ra_c = jnp.where(ma, ra, 0)
        plsc.store_scatter(arow, [ra_c, bb], onesf, mask=ma)
        rb = bb - base
        mb = (rb >= 0) & (rb < RW)
        rb_c = jnp.where(mb, rb, 0)
        plsc.store_scatter(arow, [rb_c, aa], onesf, mask=mb)

    iota16 = lax.iota(jnp.int32, 16)
    for ch in range(RW // 16):
        rv = iota16 + ch * 16
        plsc.store_scatter(arow, [rv, base + rv], onesf)

    pltpu.sync_copy(arow, a_hbm.at[pl.ds(base, RW)])


def _build_adj(edge_index):
    mesh = plsc.VectorSubcoreMesh(core_axis_name="c", subcore_axis_name="s")
    zero = jnp.zeros((RW, N), jnp.float32)
    kfn = functools.partial(
        pl.kernel,
        mesh=mesh,
        compiler_params=pltpu.CompilerParams(needs_layout_passes=False),
        out_type=jax.ShapeDtypeStruct((N, N), jnp.float32),
        scratch_types=[
            pltpu.VMEM((2 * E,), jnp.int32),
            pltpu.VMEM((RW, N), jnp.float32),
        ],
    )(_adj_body)
    return kfn(edge_index.reshape(2 * E), zero)


def _mask_body(a_ref, m_ref):
    ab = a_ref[...].astype(jnp.bfloat16)
    cmat = lax.dot_general(ab, ab, (((1,), (0,)), ((), ())),
                           preferred_element_type=jnp.float32)
    m_ref[...] = (cmat > 0.0).astype(jnp.float32)


def _mask_tc(a):
    return pl.pallas_call(
        _mask_body,
        out_shape=jax.ShapeDtypeStruct((N, N), jnp.float32),
    )(a)


_GB = 8
NH = N // 2
RWH = NH // NW


def _make_select_body(h):
    def _select_body(order_hbm, mask_hbm, h_hbm, hl_hbm, scnt_hbm,
                     morow, orow, idxb, scntv, grow, gsem):
        c = lax.axis_index("c")
        s = lax.axis_index("s")
        wid = s * NC + c
        obase = wid * RWH
        base = h * NH + obase

        pltpu.sync_copy(mask_hbm.at[pl.ds(base, RWH)], morow)
        pltpu.sync_copy(order_hbm.at[pl.ds(base * N, RWH * N)], orow)

        iota16 = lax.iota(jnp.int32, 16)
        lane0 = iota16 == 0

        def row_body(r, carry):
            selfv = jnp.broadcast_to(base + r, (16,)).astype(jnp.int32)
            idxb[pl.ds(r * M, 16)] = selfv
            idxb[pl.ds(r * M + 16, 16)] = selfv

            def chunk_cond(carry):
                k, found = carry
                return (k < N // 16) & (found < MAXN)

            def chunk_body(carry):
                k, found = carry
                ordv = orow[pl.ds(r * N + k * 16, 16)]
                rsp = jnp.broadcast_to(r, (16,)).astype(jnp.int32)
                mp = plsc.load_gather(morow, [rsp, ordv])
                mb = mp > 0.0
                cs = plsc.cumsum(mb.astype(jnp.int32))
                sel = mb & ((found + cs) <= MAXN)
                plsc.store_compressed(idxb.at[pl.ds(r * M + found, 16)], ordv,
                                      mask=sel)
                return k + 1, found + jnp.sum(sel.astype(jnp.int32))

            _, found = lax.while_loop(chunk_cond, chunk_body,
                                      (jnp.int32(0), jnp.int32(0)))
            rsp = jnp.broadcast_to(r, (16,)).astype(jnp.int32)
            fsp = jnp.broadcast_to(found, (16,)).astype(jnp.int32)
            plsc.store_scatter(scntv, [rsp], fsp, mask=lane0)
            return carry

        nb = RWH // _GB
        gd = [None] * nb
        for b in range(nb):
            lax.fori_loop(b * _GB, (b + 1) * _GB, row_body, 0)
            if b >= 1:
                gd[b - 1].wait()
                pltpu.sync_copy(
                    grow,
                    hl_hbm.at[pl.ds((obase + (b - 1) * _GB) * M, _GB * M)])
            gd[b] = pltpu.async_copy(
                h_hbm.at[idxb.at[pl.ds(b * _GB * M, _GB * M)]], grow, gsem)
        gd[nb - 1].wait()
        pltpu.sync_copy(
            grow, hl_hbm.at[pl.ds((obase + (nb - 1) * _GB) * M, _GB * M)])

        pltpu.sync_copy(scntv, scnt_hbm.at[pl.ds(obase, RWH)])

    return _select_body


def _select_gather(order, maskf, H, h):
    mesh = plsc.VectorSubcoreMesh(core_axis_name="c", subcore_axis_name="s")
    kfn = functools.partial(
        pl.kernel,
        mesh=mesh,
        compiler_params=pltpu.CompilerParams(needs_layout_passes=False),
        out_type=(
            jax.ShapeDtypeStruct((NH * M, D), jnp.float32),
            jax.ShapeDtypeStruct((NH,), jnp.int32),
        ),
        scratch_types=[
            pltpu.VMEM((RWH, N), jnp.float32),
            pltpu.VMEM((RWH * N,), jnp.int32),
            pltpu.VMEM((RWH * M + 16,), jnp.int32),
            pltpu.VMEM((RWH,), jnp.int32),
            pltpu.VMEM((_GB * M, D), jnp.float32),
            pltpu.SemaphoreType.DMA,
        ],
        name=f"select_gather_h{h}",
    )(_make_select_body(h))
    hl, scnt = kfn(order.reshape(N * N), maskf, H)
    return hl.reshape(NH, M, D), scnt


_BB = 64


def _bcast(x, shape, dims):
    return lax.broadcast_in_dim(x, shape, dims)


def _stats_body(hl_ref, scnt_ref, out_ref):
    G = hl_ref[...]
    scnt = scnt_ref[0, 0, :]
    sq = jnp.sum(G * G, axis=-1)
    inner = lax.dot_general(G, G, (((2,), (2,)), ((0,), (0,))),
                            preferred_element_type=jnp.float32)
    d2 = (_bcast(sq, (_BB, M, M), (0, 1)) + _bcast(sq, (_BB, M, M), (0, 2))
          - 2.0 * inner)
    d2 = jnp.maximum(d2, 0.0)
    dist = jnp.sqrt(d2 + 1e-12)

    mi = lax.broadcasted_iota(jnp.int32, (_BB, M, M), 1)
    ki = lax.broadcasted_iota(jnp.int32, (_BB, M, M), 2)
    sc3 = _bcast(scnt, (_BB, M, M), (0,))
    off = (mi < sc3) & (ki < sc3) & (mi != ki)
    dist_m = jnp.where(off, dist, BIG)
    pdrow = jnp.sum(jnp.where(off, dist, 0.0), axis=-1)

    keyi = lax.bitcast_convert_type(dist_m, jnp.int32)
    keyi = jnp.bitwise_or(jnp.bitwise_and(keyi, ~31), ki)
    cur = lax.bitcast_convert_type(keyi, jnp.float32)

    iota_m = lax.broadcasted_iota(jnp.int32, (_BB, M), 1)
    scb = _bcast(scnt, (_BB, M), (0,))
    valid2 = iota_m < scb

    zero = jnp.zeros((_BB, M), jnp.float32)
    s = zero
    s2 = zero
    xl = zero
    mx = zero
    nn = zero
    for t in range(KNEI):
        mn = jnp.min(cur, axis=-1)
        cur = jnp.where(cur == _bcast(mn, (_BB, M, M), (0, 1)), 1e38, cur)
        xv = lax.bitcast_convert_type(
            jnp.bitwise_and(lax.bitcast_convert_type(mn, jnp.int32), ~31),
            jnp.float32)
        wt = valid2 & ((t + 1) < scb)
        x = jnp.where(wt, xv, 0.0)
        s = s + x
        s2 = s2 + x * x
        xl = xl + x * jnp.log(x + 1e-12)
        mx = jnp.maximum(mx, x)
        nn = nn + wt.astype(jnp.float32)

    sum_knn = jnp.sum(s, axis=-1)
    n_raw = jnp.sum(nn, axis=-1)
    s2t = jnp.sum(s2, axis=-1)
    xlt = jnp.sum(xl, axis=-1)
    max_knn = jnp.max(mx, axis=-1)
    pd_t = jnp.sum(pdrow, axis=-1)

    n_k = n_raw + 1e-12
    mean_knn = sum_knn / n_k
    var = (s2t - 2.0 * mean_knn * sum_knn + mean_knn * mean_knn * n_raw) / n_k
    std_knn = jnp.sqrt(jnp.maximum(var, 0.0) + 1e-12)

    scf = scnt.astype(jnp.float32)
    n_pd = scf * scf - scf + 1e-12
    mean_pd = pd_t / n_pd

    sp = sum_knn + 1e-12
    ent = (sum_knn * jnp.log(sp) - xlt) / sp

    sigs = jnp.stack([mean_knn, max_knn, sum_knn, std_knn, mean_pd, ent],
                     axis=-1)
    gf = (scnt >= 3).astype(jnp.float32)
    out_ref[...] = sigs * _bcast(gf, (_BB, 6), (0,))


def _stats_tc(hl, scnt3):
    return pl.pallas_call(
        _stats_body,
        grid=(NH // _BB,),
        in_specs=[
            pl.BlockSpec((_BB, M, D), lambda i: (i, 0, 0)),
            pl.BlockSpec((1, 1, _BB), lambda i: (i, 0, 0)),
        ],
        out_specs=pl.BlockSpec((_BB, 6), lambda i: (i, 0)),
        out_shape=jax.ShapeDtypeStruct((NH, 6), jnp.float32),
    )(hl, scnt3)


def kernel(H, edge_index):
    order = _get_order()
    a = _build_adj(edge_index)
    maskf = _mask_tc(a)
    ordc = jnp.asarray(order)
    halves = []
    for h in range(2):
        hl, scnt = _select_gather(ordc, maskf, H, h)
        halves.append(_stats_tc(hl, scnt.reshape(NH // _BB, 1, _BB)))
    return jnp.concatenate(halves, axis=0)
